# baseline (speedup 1.0000x reference)
import functools

import jax
import jax.numpy as jnp
from jax import lax
from jax.experimental import pallas as pl
from jax.experimental.pallas import tpu as pltpu
from jax.experimental.pallas import tpu_sc as plsc

CHUNK = 128
NBUF = 2
GROUP = CHUNK * NBUF

TBLK = 512


def _transpose_pad_block(tt_ref, out_ref):
    t = tt_ref[...]
    eye = jnp.eye(t.shape[0], dtype=jnp.float32)
    tT = jax.lax.dot_general(
        t, eye, (((0,), (0,)), ((), ())), preferred_element_type=jnp.float32
    )
    out_ref[...] = jnp.concatenate([tT, jnp.zeros_like(tT)], axis=1)


@functools.lru_cache(maxsize=None)
def _make_transpose_pad(vocab: int, d: int):
    grid = (vocab + TBLK - 1) // TBLK
    return pl.pallas_call(
        _transpose_pad_block,
        grid=(grid,),
        in_specs=[pl.BlockSpec((d, TBLK), lambda j: (0, j))],
        out_specs=pl.BlockSpec((TBLK, 2 * d), lambda j: (j, 0)),
        out_shape=jax.ShapeDtypeStruct((vocab, 2 * d), jnp.float32),
    )


@functools.lru_cache(maxsize=None)
def _make_gather(n_total: int, vocab: int, dpad: int):
    info = plsc.get_sparse_core_info()
    nc, ns = info.num_cores, info.num_subcores
    nw = nc * ns
    n_per_w = n_total // nw
    n_groups = n_per_w // GROUP

    mesh = plsc.VectorSubcoreMesh(core_axis_name="c", subcore_axis_name="s")

    @functools.partial(
        pl.kernel,
        mesh=mesh,
        compiler_params=pltpu.CompilerParams(use_tc_tiling_on_sc=True),
        out_type=jax.ShapeDtypeStruct((n_total, dpad), jnp.float32),
        scratch_types=[
            pltpu.VMEM((n_per_w,), jnp.int32),
            pltpu.VMEM((2, GROUP, dpad), jnp.float32),
            pltpu.SemaphoreType.DMA,
            pltpu.SemaphoreType.DMA,
            pltpu.SemaphoreType.DMA,
            pltpu.SemaphoreType.DMA,
        ],
    )
    def gather_kernel(idx_hbm, table_hbm, out_hbm, idx_all, rows, g0, g1, s0, s1):
        wid = lax.axis_index("s") * nc + lax.axis_index("c")
        base = wid * n_per_w
        gsem = (g0, g1)
        ssem = (s0, s1)
        pltpu.sync_copy(idx_hbm.at[pl.ds(base, n_per_w)], idx_all)

        def gstart(p, g):
            for b in range(NBUF):
                pltpu.async_copy(
                    table_hbm.at[idx_all.at[pl.ds(g * GROUP + b * CHUNK, CHUNK)]],
                    rows.at[p, pl.ds(b * CHUNK, CHUNK)],
                    gsem[p],
                )

        def gwait(p):
            for b in range(NBUF):
                pltpu.make_async_copy(
                    table_hbm.at[idx_all.at[pl.ds(b * CHUNK, CHUNK)]],
                    rows.at[p, pl.ds(b * CHUNK, CHUNK)],
                    gsem[p],
                ).wait()

        def sstart(p, g):
            pltpu.async_copy(
                rows.at[p],
                out_hbm.at[pl.ds(base + g * GROUP, GROUP)],
                ssem[p],
            )

        def swait(p):
            pltpu.make_async_copy(
                rows.at[p],
                out_hbm.at[pl.ds(base, GROUP)],
                ssem[p],
            ).wait()

        def handle(g, p):
            pl.when(g > 0)(lambda: swait(1 - p))
            pl.when(g + 1 < n_groups)(lambda: gstart(1 - p, g + 1))
            gwait(p)
            sstart(p, g)

        gstart(0, 0)

        def body(i2, carry):
            handle(2 * i2, 0)
            handle(2 * i2 + 1, 1)
            return carry

        lax.fori_loop(0, n_groups // 2, body, 0)
        swait((n_groups - 1) % 2)

    return gather_kernel


def kernel(x, table):
    b, l = x.shape
    vocab, d = table.shape
    table_pad = _make_transpose_pad(vocab, d)(table.T)
    flat = x.reshape(b * l).astype(jnp.int32)
    out = _make_gather(b * l, vocab, 2 * d)(flat, table_pad)
    return out[:, :d].reshape(b, l, d)


# speedup vs baseline: 1.8530x; 1.8530x over previous
import functools

import jax
import jax.numpy as jnp
from jax import lax
from jax.experimental import pallas as pl
from jax.experimental.pallas import tpu as pltpu
from jax.experimental.pallas import tpu_sc as plsc

CHUNK = 128
NBUF = 2
GROUP = CHUNK * NBUF

TBLK = 2048


def _transpose_pad_block(tt_ref, out_ref):
    t = tt_ref[...]
    tT = t.T
    out_ref[...] = jnp.concatenate([tT, jnp.zeros_like(tT)], axis=1)


@functools.lru_cache(maxsize=None)
def _make_transpose_pad(vocab: int, d: int):
    grid = (vocab + TBLK - 1) // TBLK
    return pl.pallas_call(
        _transpose_pad_block,
        grid=(grid,),
        in_specs=[pl.BlockSpec((d, TBLK), lambda j: (0, j))],
        out_specs=pl.BlockSpec((TBLK, 2 * d), lambda j: (j, 0)),
        out_shape=jax.ShapeDtypeStruct((vocab, 2 * d), jnp.float32),
    )


@functools.lru_cache(maxsize=None)
def _make_gather(n_total: int, vocab: int, dpad: int):
    info = plsc.get_sparse_core_info()
    nc, ns = info.num_cores, info.num_subcores
    nw = nc * ns
    n_per_w = n_total // nw
    n_groups = n_per_w // GROUP

    mesh = plsc.VectorSubcoreMesh(core_axis_name="c", subcore_axis_name="s")

    @functools.partial(
        pl.kernel,
        mesh=mesh,
        compiler_params=pltpu.CompilerParams(use_tc_tiling_on_sc=True),
        out_type=jax.ShapeDtypeStruct((n_total, dpad), jnp.float32),
        scratch_types=[
            pltpu.VMEM((n_per_w,), jnp.int32),
            pltpu.VMEM((2, GROUP, dpad), jnp.float32),
            pltpu.SemaphoreType.DMA,
            pltpu.SemaphoreType.DMA,
            pltpu.SemaphoreType.DMA,
            pltpu.SemaphoreType.DMA,
        ],
    )
    def gather_kernel(idx_hbm, table_hbm, out_hbm, idx_all, rows, g0, g1, s0, s1):
        wid = lax.axis_index("s") * nc + lax.axis_index("c")
        base = wid * n_per_w
        gsem = (g0, g1)
        ssem = (s0, s1)
        pltpu.sync_copy(idx_hbm.at[pl.ds(base, n_per_w)], idx_all)

        def gstart(p, g):
            for b in range(NBUF):
                pltpu.async_copy(
                    table_hbm.at[idx_all.at[pl.ds(g * GROUP + b * CHUNK, CHUNK)]],
                    rows.at[p, pl.ds(b * CHUNK, CHUNK)],
                    gsem[p],
                )

        def gwait(p):
            for b in range(NBUF):
                pltpu.make_async_copy(
                    table_hbm.at[idx_all.at[pl.ds(b * CHUNK, CHUNK)]],
                    rows.at[p, pl.ds(b * CHUNK, CHUNK)],
                    gsem[p],
                ).wait()

        def sstart(p, g):
            pltpu.async_copy(
                rows.at[p],
                out_hbm.at[pl.ds(base + g * GROUP, GROUP)],
                ssem[p],
            )

        def swait(p):
            pltpu.make_async_copy(
                rows.at[p],
                out_hbm.at[pl.ds(base, GROUP)],
                ssem[p],
            ).wait()

        def handle(g, p):
            pl.when(g > 0)(lambda: swait(1 - p))
            pl.when(g + 1 < n_groups)(lambda: gstart(1 - p, g + 1))
            gwait(p)
            sstart(p, g)

        gstart(0, 0)

        def body(i2, carry):
            handle(2 * i2, 0)
            handle(2 * i2 + 1, 1)
            return carry

        lax.fori_loop(0, n_groups // 2, body, 0)
        swait((n_groups - 1) % 2)

    return gather_kernel


def kernel(x, table):
    b, l = x.shape
    vocab, d = table.shape
    table_pad = _make_transpose_pad(vocab, d)(table.T)
    flat = x.reshape(b * l).astype(jnp.int32)
    out = _make_gather(b * l, vocab, 2 * d)(flat, table_pad)
    return out[:, :d].reshape(b, l, d)


# speedup vs baseline: 2.3372x; 1.2613x over previous
import functools

import jax
import jax.numpy as jnp
from jax import lax
from jax.experimental import pallas as pl
from jax.experimental.pallas import tpu as pltpu
from jax.experimental.pallas import tpu_sc as plsc

CHUNK = 128
NBUF = 2
GROUP = CHUNK * NBUF

TBLK = 8192


def _transpose_pad_block(tt_ref, out_ref):
    t = tt_ref[...]
    tT = t.T
    out_ref[...] = jnp.concatenate([tT, jnp.zeros_like(tT)], axis=1)


@functools.lru_cache(maxsize=None)
def _make_transpose_pad(vocab: int, d: int):
    grid = (vocab + TBLK - 1) // TBLK
    return pl.pallas_call(
        _transpose_pad_block,
        grid=(grid,),
        in_specs=[pl.BlockSpec((d, TBLK), lambda j: (0, j))],
        out_specs=pl.BlockSpec((TBLK, 2 * d), lambda j: (j, 0)),
        out_shape=jax.ShapeDtypeStruct((vocab, 2 * d), jnp.float32),
    )


@functools.lru_cache(maxsize=None)
def _make_gather(n_total: int, vocab: int, dpad: int):
    info = plsc.get_sparse_core_info()
    nc, ns = info.num_cores, info.num_subcores
    nw = nc * ns
    n_per_w = n_total // nw
    n_groups = n_per_w // GROUP

    mesh = plsc.VectorSubcoreMesh(core_axis_name="c", subcore_axis_name="s")

    @functools.partial(
        pl.kernel,
        mesh=mesh,
        compiler_params=pltpu.CompilerParams(use_tc_tiling_on_sc=True),
        out_type=jax.ShapeDtypeStruct((n_total, dpad), jnp.float32),
        scratch_types=[
            pltpu.VMEM((n_per_w,), jnp.int32),
            pltpu.VMEM((2, GROUP, dpad), jnp.float32),
            pltpu.SemaphoreType.DMA,
            pltpu.SemaphoreType.DMA,
            pltpu.SemaphoreType.DMA,
            pltpu.SemaphoreType.DMA,
        ],
    )
    def gather_kernel(idx_hbm, table_hbm, out_hbm, idx_all, rows, g0, g1, s0, s1):
        wid = lax.axis_index("s") * nc + lax.axis_index("c")
        base = wid * n_per_w
        gsem = (g0, g1)
        ssem = (s0, s1)
        pltpu.sync_copy(idx_hbm.at[pl.ds(base, n_per_w)], idx_all)

        def gstart(p, g):
            for b in range(NBUF):
                pltpu.async_copy(
                    table_hbm.at[idx_all.at[pl.ds(g * GROUP + b * CHUNK, CHUNK)]],
                    rows.at[p, pl.ds(b * CHUNK, CHUNK)],
                    gsem[p],
                )

        def gwait(p):
            for b in range(NBUF):
                pltpu.make_async_copy(
                    table_hbm.at[idx_all.at[pl.ds(b * CHUNK, CHUNK)]],
                    rows.at[p, pl.ds(b * CHUNK, CHUNK)],
                    gsem[p],
                ).wait()

        def sstart(p, g):
            pltpu.async_copy(
                rows.at[p],
                out_hbm.at[pl.ds(base + g * GROUP, GROUP)],
                ssem[p],
            )

        def swait(p):
            pltpu.make_async_copy(
                rows.at[p],
                out_hbm.at[pl.ds(base, GROUP)],
                ssem[p],
            ).wait()

        def handle(g, p):
            pl.when(g > 0)(lambda: swait(1 - p))
            pl.when(g + 1 < n_groups)(lambda: gstart(1 - p, g + 1))
            gwait(p)
            sstart(p, g)

        gstart(0, 0)

        def body(i2, carry):
            handle(2 * i2, 0)
            handle(2 * i2 + 1, 1)
            return carry

        lax.fori_loop(0, n_groups // 2, body, 0)
        swait((n_groups - 1) % 2)

    return gather_kernel


def kernel(x, table):
    b, l = x.shape
    vocab, d = table.shape
    table_pad = _make_transpose_pad(vocab, d)(table.T)
    flat = x.reshape(b * l).astype(jnp.int32)
    out = _make_gather(b * l, vocab, 2 * d)(flat, table_pad)
    return out[:, :d].reshape(b, l, d)


# speedup vs baseline: 2.4003x; 1.0270x over previous
import functools

import jax
import jax.numpy as jnp
from jax import lax
from jax.experimental import pallas as pl
from jax.experimental.pallas import tpu as pltpu
from jax.experimental.pallas import tpu_sc as plsc

CHUNK = 128
NBUF = 2
GROUP = CHUNK * NBUF

TBLK = 16384


def _transpose_pad_block(tt_ref, out_ref):
    t = tt_ref[...]
    tT = t.T
    out_ref[...] = jnp.concatenate([tT, jnp.zeros_like(tT)], axis=1)


@functools.lru_cache(maxsize=None)
def _make_transpose_pad(vocab: int, d: int):
    grid = (vocab + TBLK - 1) // TBLK
    return pl.pallas_call(
        _transpose_pad_block,
        grid=(grid,),
        in_specs=[pl.BlockSpec((d, TBLK), lambda j: (0, j))],
        out_specs=pl.BlockSpec((TBLK, 2 * d), lambda j: (j, 0)),
        out_shape=jax.ShapeDtypeStruct((vocab, 2 * d), jnp.float32),
    )


@functools.lru_cache(maxsize=None)
def _make_gather(n_total: int, vocab: int, dpad: int):
    info = plsc.get_sparse_core_info()
    nc, ns = info.num_cores, info.num_subcores
    nw = nc * ns
    n_per_w = n_total // nw
    n_groups = n_per_w // GROUP

    mesh = plsc.VectorSubcoreMesh(core_axis_name="c", subcore_axis_name="s")

    @functools.partial(
        pl.kernel,
        mesh=mesh,
        compiler_params=pltpu.CompilerParams(use_tc_tiling_on_sc=True),
        out_type=jax.ShapeDtypeStruct((n_total, dpad), jnp.float32),
        scratch_types=[
            pltpu.VMEM((n_per_w,), jnp.int32),
            pltpu.VMEM((2, GROUP, dpad), jnp.float32),
            pltpu.SemaphoreType.DMA,
            pltpu.SemaphoreType.DMA,
            pltpu.SemaphoreType.DMA,
            pltpu.SemaphoreType.DMA,
        ],
    )
    def gather_kernel(idx_hbm, table_hbm, out_hbm, idx_all, rows, g0, g1, s0, s1):
        wid = lax.axis_index("s") * nc + lax.axis_index("c")
        base = wid * n_per_w
        gsem = (g0, g1)
        ssem = (s0, s1)
        pltpu.sync_copy(idx_hbm.at[pl.ds(base, n_per_w)], idx_all)

        def gstart(p, g):
            for b in range(NBUF):
                pltpu.async_copy(
                    table_hbm.at[idx_all.at[pl.ds(g * GROUP + b * CHUNK, CHUNK)]],
                    rows.at[p, pl.ds(b * CHUNK, CHUNK)],
                    gsem[p],
                )

        def gwait(p):
            for b in range(NBUF):
                pltpu.make_async_copy(
                    table_hbm.at[idx_all.at[pl.ds(b * CHUNK, CHUNK)]],
                    rows.at[p, pl.ds(b * CHUNK, CHUNK)],
                    gsem[p],
                ).wait()

        def sstart(p, g):
            pltpu.async_copy(
                rows.at[p],
                out_hbm.at[pl.ds(base + g * GROUP, GROUP)],
                ssem[p],
            )

        def swait(p):
            pltpu.make_async_copy(
                rows.at[p],
                out_hbm.at[pl.ds(base, GROUP)],
                ssem[p],
            ).wait()

        def handle(g, p):
            pl.when(g > 0)(lambda: swait(1 - p))
            pl.when(g + 1 < n_groups)(lambda: gstart(1 - p, g + 1))
            gwait(p)
            sstart(p, g)

        gstart(0, 0)

        def body(i2, carry):
            handle(2 * i2, 0)
            handle(2 * i2 + 1, 1)
            return carry

        lax.fori_loop(0, n_groups // 2, body, 0)
        swait((n_groups - 1) % 2)

    return gather_kernel


def kernel(x, table):
    b, l = x.shape
    vocab, d = table.shape
    table_pad = _make_transpose_pad(vocab, d)(table.T)
    flat = x.reshape(b * l).astype(jnp.int32)
    out = _make_gather(b * l, vocab, 2 * d)(flat, table_pad)
    return out[:, :d].reshape(b, l, d)


# speedup vs baseline: 2.4286x; 1.0118x over previous
import functools

import jax
import jax.numpy as jnp
from jax import lax
from jax.experimental import pallas as pl
from jax.experimental.pallas import tpu as pltpu
from jax.experimental.pallas import tpu_sc as plsc

CHUNK = 128
NBUF = 2
GROUP = CHUNK * NBUF

TBLK = 32768


def _transpose_pad_block(tt_ref, out_ref):
    t = tt_ref[...]
    tT = t.T
    out_ref[...] = jnp.concatenate([tT, jnp.zeros_like(tT)], axis=1)


@functools.lru_cache(maxsize=None)
def _make_transpose_pad(vocab: int, d: int):
    grid = (vocab + TBLK - 1) // TBLK
    return pl.pallas_call(
        _transpose_pad_block,
        grid=(grid,),
        in_specs=[pl.BlockSpec((d, TBLK), lambda j: (0, j))],
        out_specs=pl.BlockSpec((TBLK, 2 * d), lambda j: (j, 0)),
        out_shape=jax.ShapeDtypeStruct((vocab, 2 * d), jnp.float32),
    )


@functools.lru_cache(maxsize=None)
def _make_gather(n_total: int, vocab: int, dpad: int):
    info = plsc.get_sparse_core_info()
    nc, ns = info.num_cores, info.num_subcores
    nw = nc * ns
    n_per_w = n_total // nw
    n_groups = n_per_w // GROUP

    mesh = plsc.VectorSubcoreMesh(core_axis_name="c", subcore_axis_name="s")

    @functools.partial(
        pl.kernel,
        mesh=mesh,
        compiler_params=pltpu.CompilerParams(use_tc_tiling_on_sc=True),
        out_type=jax.ShapeDtypeStruct((n_total, dpad), jnp.float32),
        scratch_types=[
            pltpu.VMEM((n_per_w,), jnp.int32),
            pltpu.VMEM((2, GROUP, dpad), jnp.float32),
            pltpu.SemaphoreType.DMA,
            pltpu.SemaphoreType.DMA,
            pltpu.SemaphoreType.DMA,
            pltpu.SemaphoreType.DMA,
        ],
    )
    def gather_kernel(idx_hbm, table_hbm, out_hbm, idx_all, rows, g0, g1, s0, s1):
        wid = lax.axis_index("s") * nc + lax.axis_index("c")
        base = wid * n_per_w
        gsem = (g0, g1)
        ssem = (s0, s1)
        pltpu.sync_copy(idx_hbm.at[pl.ds(base, n_per_w)], idx_all)

        def gstart(p, g):
            for b in range(NBUF):
                pltpu.async_copy(
                    table_hbm.at[idx_all.at[pl.ds(g * GROUP + b * CHUNK, CHUNK)]],
                    rows.at[p, pl.ds(b * CHUNK, CHUNK)],
                    gsem[p],
                )

        def gwait(p):
            for b in range(NBUF):
                pltpu.make_async_copy(
                    table_hbm.at[idx_all.at[pl.ds(b * CHUNK, CHUNK)]],
                    rows.at[p, pl.ds(b * CHUNK, CHUNK)],
                    gsem[p],
                ).wait()

        def sstart(p, g):
            pltpu.async_copy(
                rows.at[p],
                out_hbm.at[pl.ds(base + g * GROUP, GROUP)],
                ssem[p],
            )

        def swait(p):
            pltpu.make_async_copy(
                rows.at[p],
                out_hbm.at[pl.ds(base, GROUP)],
                ssem[p],
            ).wait()

        def handle(g, p):
            pl.when(g > 0)(lambda: swait(1 - p))
            pl.when(g + 1 < n_groups)(lambda: gstart(1 - p, g + 1))
            gwait(p)
            sstart(p, g)

        gstart(0, 0)

        def body(i2, carry):
            handle(2 * i2, 0)
            handle(2 * i2 + 1, 1)
            return carry

        lax.fori_loop(0, n_groups // 2, body, 0)
        swait((n_groups - 1) % 2)

    return gather_kernel


def kernel(x, table):
    b, l = x.shape
    vocab, d = table.shape
    table_pad = _make_transpose_pad(vocab, d)(table.T)
    flat = x.reshape(b * l).astype(jnp.int32)
    out = _make_gather(b * l, vocab, 2 * d)(flat, table_pad)
    return out[:, :d].reshape(b, l, d)


# speedup vs baseline: 2.4316x; 1.0012x over previous
import functools

import jax
import jax.numpy as jnp
from jax import lax
from jax.experimental import pallas as pl
from jax.experimental.pallas import tpu as pltpu
from jax.experimental.pallas import tpu_sc as plsc

CHUNK = 128
NBUF = 2
GROUP = CHUNK * NBUF

TBLK = 32768


def _transpose_pad_block(tt_ref, out_ref):
    t = tt_ref[...]
    tT = t.T
    out_ref[...] = jnp.concatenate([tT, jnp.zeros_like(tT)], axis=1)


@functools.lru_cache(maxsize=None)
def _make_transpose_pad(vocab: int, d: int):
    grid = (vocab + TBLK - 1) // TBLK
    return pl.pallas_call(
        _transpose_pad_block,
        grid=(grid,),
        in_specs=[pl.BlockSpec((d, TBLK), lambda j: (0, j))],
        out_specs=pl.BlockSpec((TBLK, 2 * d), lambda j: (j, 0)),
        out_shape=jax.ShapeDtypeStruct((vocab, 2 * d), jnp.float32),
    )


@functools.lru_cache(maxsize=None)
def _make_gather(n_total: int, vocab: int, dpad: int):
    info = plsc.get_sparse_core_info()
    nc, ns = info.num_cores, info.num_subcores
    nw = nc * ns
    n_per_w = n_total // nw
    n_groups = n_per_w // GROUP

    mesh = plsc.VectorSubcoreMesh(core_axis_name="c", subcore_axis_name="s")

    @functools.partial(
        pl.kernel,
        mesh=mesh,
        compiler_params=pltpu.CompilerParams(use_tc_tiling_on_sc=True),
        out_type=jax.ShapeDtypeStruct((n_total, dpad), jnp.float32),
        scratch_types=[
            pltpu.VMEM((n_per_w,), jnp.int32),
            pltpu.VMEM((3, GROUP, dpad), jnp.float32),
            pltpu.SemaphoreType.DMA,
            pltpu.SemaphoreType.DMA,
            pltpu.SemaphoreType.DMA,
            pltpu.SemaphoreType.DMA,
            pltpu.SemaphoreType.DMA,
            pltpu.SemaphoreType.DMA,
        ],
    )
    def gather_kernel(
        idx_hbm, table_hbm, out_hbm, idx_all, rows, g0, g1, g2, s0, s1, s2
    ):
        wid = lax.axis_index("s") * nc + lax.axis_index("c")
        base = wid * n_per_w
        gsem = (g0, g1, g2)
        ssem = (s0, s1, s2)
        pltpu.sync_copy(idx_hbm.at[pl.ds(base, n_per_w)], idx_all)

        def gstart(p, g):
            for b in range(NBUF):
                pltpu.async_copy(
                    table_hbm.at[idx_all.at[pl.ds(g * GROUP + b * CHUNK, CHUNK)]],
                    rows.at[p, pl.ds(b * CHUNK, CHUNK)],
                    gsem[p],
                )

        def gwait(p):
            for b in range(NBUF):
                pltpu.make_async_copy(
                    table_hbm.at[idx_all.at[pl.ds(b * CHUNK, CHUNK)]],
                    rows.at[p, pl.ds(b * CHUNK, CHUNK)],
                    gsem[p],
                ).wait()

        def sstart(p, g):
            pltpu.async_copy(
                rows.at[p],
                out_hbm.at[pl.ds(base + g * GROUP, GROUP)],
                ssem[p],
            )

        def swait(p):
            pltpu.make_async_copy(
                rows.at[p],
                out_hbm.at[pl.ds(base, GROUP)],
                ssem[p],
            ).wait()

        def handle(g, p):
            gwait(p)
            sstart(p, g)
            pv = (p + 2) % 3
            pl.when(g >= 1)(lambda: swait(pv))
            pl.when(g + 2 < n_groups)(lambda: gstart(pv, g + 2))

        assert n_groups % 3 == 1
        gstart(0, 0)
        gstart(1, 1)

        def body(i3, carry):
            handle(3 * i3, 0)
            handle(3 * i3 + 1, 1)
            handle(3 * i3 + 2, 2)
            return carry

        lax.fori_loop(0, n_groups // 3, body, 0)
        handle(n_groups - 1, (n_groups - 1) % 3)
        swait((n_groups - 1) % 3)

    return gather_kernel


def kernel(x, table):
    b, l = x.shape
    vocab, d = table.shape
    table_pad = _make_transpose_pad(vocab, d)(table.T)
    flat = x.reshape(b * l).astype(jnp.int32)
    out = _make_gather(b * l, vocab, 2 * d)(flat, table_pad)
    return out[:, :d].reshape(b, l, d)


# speedup vs baseline: 2.4358x; 1.0017x over previous
import functools

import jax
import jax.numpy as jnp
from jax import lax
from jax.experimental import pallas as pl
from jax.experimental.pallas import tpu as pltpu
from jax.experimental.pallas import tpu_sc as plsc

CHUNK = 256
NBUF = 1
GROUP = CHUNK * NBUF

TBLK = 32768


def _transpose_pad_block(tt_ref, out_ref):
    t = tt_ref[...]
    tT = t.T
    out_ref[...] = jnp.concatenate([tT, jnp.zeros_like(tT)], axis=1)


@functools.lru_cache(maxsize=None)
def _make_transpose_pad(vocab: int, d: int):
    grid = (vocab + TBLK - 1) // TBLK
    return pl.pallas_call(
        _transpose_pad_block,
        grid=(grid,),
        in_specs=[pl.BlockSpec((d, TBLK), lambda j: (0, j))],
        out_specs=pl.BlockSpec((TBLK, 2 * d), lambda j: (j, 0)),
        out_shape=jax.ShapeDtypeStruct((vocab, 2 * d), jnp.float32),
    )


@functools.lru_cache(maxsize=None)
def _make_gather(n_total: int, vocab: int, dpad: int):
    info = plsc.get_sparse_core_info()
    nc, ns = info.num_cores, info.num_subcores
    nw = nc * ns
    n_per_w = n_total // nw
    n_groups = n_per_w // GROUP

    mesh = plsc.VectorSubcoreMesh(core_axis_name="c", subcore_axis_name="s")

    @functools.partial(
        pl.kernel,
        mesh=mesh,
        compiler_params=pltpu.CompilerParams(use_tc_tiling_on_sc=True),
        out_type=jax.ShapeDtypeStruct((n_total, dpad), jnp.float32),
        scratch_types=[
            pltpu.VMEM((n_per_w,), jnp.int32),
            pltpu.VMEM((3, GROUP, dpad), jnp.float32),
            pltpu.SemaphoreType.DMA,
            pltpu.SemaphoreType.DMA,
            pltpu.SemaphoreType.DMA,
            pltpu.SemaphoreType.DMA,
            pltpu.SemaphoreType.DMA,
            pltpu.SemaphoreType.DMA,
        ],
    )
    def gather_kernel(
        idx_hbm, table_hbm, out_hbm, idx_all, rows, g0, g1, g2, s0, s1, s2
    ):
        wid = lax.axis_index("s") * nc + lax.axis_index("c")
        base = wid * n_per_w
        gsem = (g0, g1, g2)
        ssem = (s0, s1, s2)
        pltpu.sync_copy(idx_hbm.at[pl.ds(base, n_per_w)], idx_all)

        def gstart(p, g):
            for b in range(NBUF):
                pltpu.async_copy(
                    table_hbm.at[idx_all.at[pl.ds(g * GROUP + b * CHUNK, CHUNK)]],
                    rows.at[p, pl.ds(b * CHUNK, CHUNK)],
                    gsem[p],
                )

        def gwait(p):
            for b in range(NBUF):
                pltpu.make_async_copy(
                    table_hbm.at[idx_all.at[pl.ds(b * CHUNK, CHUNK)]],
                    rows.at[p, pl.ds(b * CHUNK, CHUNK)],
                    gsem[p],
                ).wait()

        def sstart(p, g):
            pltpu.async_copy(
                rows.at[p],
                out_hbm.at[pl.ds(base + g * GROUP, GROUP)],
                ssem[p],
            )

        def swait(p):
            pltpu.make_async_copy(
                rows.at[p],
                out_hbm.at[pl.ds(base, GROUP)],
                ssem[p],
            ).wait()

        def handle(g, p):
            gwait(p)
            sstart(p, g)
            pv = (p + 2) % 3
            pl.when(g >= 1)(lambda: swait(pv))
            pl.when(g + 2 < n_groups)(lambda: gstart(pv, g + 2))

        assert n_groups % 3 == 1
        gstart(0, 0)
        gstart(1, 1)

        def body(i3, carry):
            handle(3 * i3, 0)
            handle(3 * i3 + 1, 1)
            handle(3 * i3 + 2, 2)
            return carry

        lax.fori_loop(0, n_groups // 3, body, 0)
        handle(n_groups - 1, (n_groups - 1) % 3)
        swait((n_groups - 1) % 3)

    return gather_kernel


def kernel(x, table):
    b, l = x.shape
    vocab, d = table.shape
    table_pad = _make_transpose_pad(vocab, d)(table.T)
    flat = x.reshape(b * l).astype(jnp.int32)
    out = _make_gather(b * l, vocab, 2 * d)(flat, table_pad)
    return out[:, :d].reshape(b, l, d)
